# baseline (speedup 1.0000x reference)
import functools

import jax
import jax.numpy as jnp
from jax import lax
from jax.experimental import pallas as pl
from jax.experimental.pallas import tpu as pltpu
from jax.experimental.pallas import tpu_sc as plsc

N = 10000
E = 320000
D_IN = 128
W_AUG = 136
NC, NS = 2, 16
NW = NC * NS
EDGES_PER_TILE = E // NW
CHUNK = 80
NCHUNK = EDGES_PER_TILE // CHUNK
PHASE_A = 63
ROWS_PER_SUBCORE = N // NS

_sc_mesh = plsc.VectorSubcoreMesh(core_axis_name="c", subcore_axis_name="s")


@functools.partial(
    pl.kernel,
    out_type=jax.ShapeDtypeStruct((NC * N, W_AUG), jnp.float32),
    mesh=_sc_mesh,
    scratch_types=[
        pltpu.VMEM_SHARED((N, W_AUG), jnp.float32),
        pltpu.VMEM((CHUNK, W_AUG), jnp.float32),
        pltpu.VMEM((CHUNK, W_AUG), jnp.float32),
        pltpu.VMEM((CHUNK, W_AUG), jnp.float32),
        pltpu.VMEM((PHASE_A, CHUNK), jnp.int32),
        pltpu.VMEM((PHASE_A, CHUNK), jnp.int32),
        pltpu.SemaphoreType.DMA,
        pltpu.SemaphoreType.DMA,
        pltpu.SemaphoreType.DMA,
    ],
    compiler_params=pltpu.CompilerParams(use_tc_tiling_on_sc=False),
)
def _sc_scatter_pass(table, gidx, sidx, zeros, out, acc, rows0, rows1, rows2,
                     gia, sia, sem0, sem1, sem2):
    c = lax.axis_index("c")
    s = lax.axis_index("s")
    wid = c * NS + s
    rows_b = (rows0, rows1, rows2)
    sem_b = (sem0, sem1, sem2)
    pltpu.sync_copy(zeros, acc.at[pl.ds(s * ROWS_PER_SUBCORE, ROWS_PER_SUBCORE)])
    pltpu.sync_copy(gidx.at[wid, pl.ds(0, PHASE_A)], gia)
    pltpu.sync_copy(sidx.at[wid, pl.ds(0, PHASE_A)], sia)
    plsc.subcore_barrier()

    for off, nch in ((0, PHASE_A), (PHASE_A, NCHUNK - PHASE_A)):
        if off:
            pltpu.sync_copy(gidx.at[wid, pl.ds(off, nch)], gia.at[pl.ds(0, nch)])
            pltpu.sync_copy(sidx.at[wid, pl.ds(off, nch)], sia.at[pl.ds(0, nch)])
        pltpu.async_copy(table.at[gia.at[0]], rows0, sem0)
        pltpu.async_copy(table.at[gia.at[1]], rows1, sem1)

        def tri_body(i, carry, nch=nch):
            c0 = 3 * i
            pltpu.async_copy(table.at[gia.at[c0 + 2]], rows2, sem2)
            for k in range(3):
                cj = c0 + k
                pltpu.make_async_copy(table.at[gia.at[cj]], rows_b[k], sem_b[k]).wait()
                pltpu.sync_copy(rows_b[k], acc.at[sia.at[cj]], add=True)
                if k < 2:
                    @pl.when(cj + 3 < nch)
                    def _():
                        pltpu.async_copy(table.at[gia.at[cj + 3]], rows_b[k], sem_b[k])
            return carry

        lax.fori_loop(0, nch // 3, tri_body, 0)
        for j in range(nch - nch % 3, nch):
            b = j % 3
            pltpu.make_async_copy(table.at[gia.at[j]], rows_b[b], sem_b[b]).wait()
            pltpu.sync_copy(rows_b[b], acc.at[sia.at[j]], add=True)

    plsc.subcore_barrier()
    row0 = s * ROWS_PER_SUBCORE
    pltpu.sync_copy(
        acc.at[pl.ds(row0, ROWS_PER_SUBCORE)],
        out.at[pl.ds(c * N + row0, ROWS_PER_SUBCORE)],
    )


def _mid_body(p_ref, o_ref):
    p = p_ref[...]
    ssum = p[0] + p[1]
    deg = ssum[:, D_IN]
    inv = jnp.where(deg > 0, 1.0 / deg, 0.0)
    he = ssum[:, :D_IN] * inv[:, None]
    bm = he.shape[0]
    pad = jnp.concatenate(
        [jnp.ones((bm, 1), jnp.float32), jnp.zeros((bm, W_AUG - D_IN - 1), jnp.float32)],
        axis=1,
    )
    o_ref[...] = jnp.concatenate([he, pad], axis=1)


def _final_body(q_ref, whg_ref, bhg_ref, wl_ref, bl_ref, o_ref):
    q = q_ref[...]
    ssum = q[0] + q[1]
    deg = ssum[:, D_IN]
    inv = jnp.where(deg > 0, 1.0 / deg, 0.0)
    h = ssum[:, :D_IN] * inv[:, None]
    h = lax.dot_general(
        h, whg_ref[...], (((1,), (1,)), ((), ())),
        preferred_element_type=jnp.float32,
    ) + bhg_ref[...]
    y = lax.dot_general(
        h, wl_ref[...], (((1,), (1,)), ((), ())),
        preferred_element_type=jnp.float32,
    ) + bl_ref[...]
    m = jnp.max(y, axis=1, keepdims=True)
    e = jnp.exp(y - m)
    o_ref[...] = e / jnp.sum(e, axis=1, keepdims=True)


_BM = 1000


def kernel(X, edge_index, W_hg, b_hg, W_lin, b_lin):
    node_idx = edge_index[0].reshape(NW, NCHUNK, CHUNK)
    he_idx = edge_index[1].reshape(NW, NCHUNK, CHUNK)
    zeros = jnp.zeros((ROWS_PER_SUBCORE, W_AUG), jnp.float32)
    x_aug = jnp.concatenate(
        [X, jnp.ones((N, 1), jnp.float32),
         jnp.zeros((N, W_AUG - D_IN - 1), jnp.float32)], axis=1)

    he_part = _sc_scatter_pass(x_aug, node_idx, he_idx, zeros)
    he_part = he_part.reshape(NC, N, W_AUG)

    he_aug = pl.pallas_call(
        _mid_body,
        grid=(N // _BM,),
        in_specs=[pl.BlockSpec((NC, _BM, W_AUG), lambda i: (0, i, 0))],
        out_specs=pl.BlockSpec((_BM, W_AUG), lambda i: (i, 0)),
        out_shape=jax.ShapeDtypeStruct((N, W_AUG), jnp.float32),
    )(he_part)

    out_part = _sc_scatter_pass(he_aug, he_idx, node_idx, zeros)
    out_part = out_part.reshape(NC, N, W_AUG)

    y = pl.pallas_call(
        _final_body,
        grid=(N // _BM,),
        in_specs=[
            pl.BlockSpec((NC, _BM, W_AUG), lambda i: (0, i, 0)),
            pl.BlockSpec((D_IN, D_IN), lambda i: (0, 0)),
            pl.BlockSpec((1, D_IN), lambda i: (0, 0)),
            pl.BlockSpec((D_IN, D_IN), lambda i: (0, 0)),
            pl.BlockSpec((1, D_IN), lambda i: (0, 0)),
        ],
        out_specs=pl.BlockSpec((_BM, D_IN), lambda i: (i, 0)),
        out_shape=jax.ShapeDtypeStruct((N, D_IN), jnp.float32),
    )(out_part, W_hg, b_hg.reshape(1, D_IN), W_lin, b_lin.reshape(1, D_IN))

    return y


# speedup vs baseline: 1.0132x; 1.0132x over previous
import functools

import jax
import jax.numpy as jnp
from jax import lax
from jax.experimental import pallas as pl
from jax.experimental.pallas import tpu as pltpu
from jax.experimental.pallas import tpu_sc as plsc

N = 10000
E = 320000
D_IN = 128
W_AUG = 136
NC, NS = 2, 16
NW = NC * NS
EDGES_PER_TILE = E // NW
CHUNK = 80
NCHUNK = EDGES_PER_TILE // CHUNK
PHASE_A = 63
ROWS_PER_SUBCORE = N // NS

_sc_mesh = plsc.VectorSubcoreMesh(core_axis_name="c", subcore_axis_name="s")


@functools.partial(
    pl.kernel,
    out_type=jax.ShapeDtypeStruct((NC * N, W_AUG), jnp.float32),
    mesh=_sc_mesh,
    scratch_types=[
        pltpu.VMEM_SHARED((N, W_AUG), jnp.float32),
        pltpu.VMEM((CHUNK, W_AUG), jnp.float32),
        pltpu.VMEM((CHUNK, W_AUG), jnp.float32),
        pltpu.VMEM((CHUNK, W_AUG), jnp.float32),
        pltpu.VMEM((PHASE_A, CHUNK), jnp.int32),
        pltpu.VMEM((PHASE_A, CHUNK), jnp.int32),
        pltpu.SemaphoreType.DMA,
        pltpu.SemaphoreType.DMA,
        pltpu.SemaphoreType.DMA,
    ],
    compiler_params=pltpu.CompilerParams(use_tc_tiling_on_sc=False),
)
def _sc_scatter_pass(table, gidx, sidx, zeros, out, acc, rows0, rows1, rows2,
                     gia, sia, sem0, sem1, sem2):
    c = lax.axis_index("c")
    s = lax.axis_index("s")
    wid = c * NS + s
    rows_b = (rows0, rows1, rows2)
    sem_b = (sem0, sem1, sem2)
    pltpu.sync_copy(zeros, acc.at[pl.ds(s * ROWS_PER_SUBCORE, ROWS_PER_SUBCORE)])
    pltpu.sync_copy(gidx.at[wid, pl.ds(0, PHASE_A)], gia)
    pltpu.sync_copy(sidx.at[wid, pl.ds(0, PHASE_A)], sia)
    plsc.subcore_barrier()

    for off, nch in ((0, PHASE_A), (PHASE_A, NCHUNK - PHASE_A)):
        if off:
            pltpu.sync_copy(gidx.at[wid, pl.ds(off, nch)], gia.at[pl.ds(0, nch)])
            pltpu.sync_copy(sidx.at[wid, pl.ds(off, nch)], sia.at[pl.ds(0, nch)])
        pltpu.async_copy(table.at[gia.at[0]], rows0, sem0)
        pltpu.async_copy(table.at[gia.at[1]], rows1, sem1)

        def tri_body(i, carry, nch=nch):
            c0 = 3 * i
            pltpu.async_copy(table.at[gia.at[c0 + 2]], rows2, sem2)
            for k in range(3):
                cj = c0 + k
                pltpu.make_async_copy(table.at[gia.at[cj]], rows_b[k], sem_b[k]).wait()
                pltpu.sync_copy(rows_b[k], acc.at[sia.at[cj]], add=True)
                if k < 2:
                    @pl.when(cj + 3 < nch)
                    def _():
                        pltpu.async_copy(table.at[gia.at[cj + 3]], rows_b[k], sem_b[k])
            return carry

        lax.fori_loop(0, nch // 3, tri_body, 0)
        for j in range(nch - nch % 3, nch):
            b = j % 3
            pltpu.make_async_copy(table.at[gia.at[j]], rows_b[b], sem_b[b]).wait()
            pltpu.sync_copy(rows_b[b], acc.at[sia.at[j]], add=True)

    plsc.subcore_barrier()
    row0 = s * ROWS_PER_SUBCORE
    pltpu.sync_copy(
        acc.at[pl.ds(row0, ROWS_PER_SUBCORE)],
        out.at[pl.ds(c * N + row0, ROWS_PER_SUBCORE)],
    )


def _mid_body(p_ref, o_ref):
    p = p_ref[...]
    ssum = p[0] + p[1]
    deg = ssum[:, D_IN]
    inv = jnp.where(deg > 0, 1.0 / deg, 0.0)
    he = ssum[:, :D_IN] * inv[:, None]
    bm = he.shape[0]
    pad = jnp.concatenate(
        [jnp.ones((bm, 1), jnp.float32), jnp.zeros((bm, W_AUG - D_IN - 1), jnp.float32)],
        axis=1,
    )
    o_ref[...] = jnp.concatenate([he, pad], axis=1)


def _final_body(q_ref, whg_ref, bhg_ref, wl_ref, bl_ref, o_ref):
    q = q_ref[...]
    ssum = q[0] + q[1]
    deg = ssum[:, D_IN]
    inv = jnp.where(deg > 0, 1.0 / deg, 0.0)
    h = ssum[:, :D_IN] * inv[:, None]
    h = lax.dot_general(
        h, whg_ref[...], (((1,), (1,)), ((), ())),
        preferred_element_type=jnp.float32,
    ) + bhg_ref[...]
    y = lax.dot_general(
        h, wl_ref[...], (((1,), (1,)), ((), ())),
        preferred_element_type=jnp.float32,
    ) + bl_ref[...]
    m = jnp.max(y, axis=1, keepdims=True)
    e = jnp.exp(y - m)
    o_ref[...] = e / jnp.sum(e, axis=1, keepdims=True)


_BM = 2000


def kernel(X, edge_index, W_hg, b_hg, W_lin, b_lin):
    node_idx = edge_index[0].reshape(NW, NCHUNK, CHUNK)
    he_idx = edge_index[1].reshape(NW, NCHUNK, CHUNK)
    zeros = jnp.zeros((ROWS_PER_SUBCORE, W_AUG), jnp.float32)
    x_aug = jnp.concatenate(
        [X, jnp.ones((N, 1), jnp.float32),
         jnp.zeros((N, W_AUG - D_IN - 1), jnp.float32)], axis=1)

    he_part = _sc_scatter_pass(x_aug, node_idx, he_idx, zeros)
    he_part = he_part.reshape(NC, N, W_AUG)

    he_aug = pl.pallas_call(
        _mid_body,
        grid=(N // _BM,),
        in_specs=[pl.BlockSpec((NC, _BM, W_AUG), lambda i: (0, i, 0))],
        out_specs=pl.BlockSpec((_BM, W_AUG), lambda i: (i, 0)),
        out_shape=jax.ShapeDtypeStruct((N, W_AUG), jnp.float32),
    )(he_part)

    out_part = _sc_scatter_pass(he_aug, he_idx, node_idx, zeros)
    out_part = out_part.reshape(NC, N, W_AUG)

    y = pl.pallas_call(
        _final_body,
        grid=(N // _BM,),
        in_specs=[
            pl.BlockSpec((NC, _BM, W_AUG), lambda i: (0, i, 0)),
            pl.BlockSpec((D_IN, D_IN), lambda i: (0, 0)),
            pl.BlockSpec((1, D_IN), lambda i: (0, 0)),
            pl.BlockSpec((D_IN, D_IN), lambda i: (0, 0)),
            pl.BlockSpec((1, D_IN), lambda i: (0, 0)),
        ],
        out_specs=pl.BlockSpec((_BM, D_IN), lambda i: (i, 0)),
        out_shape=jax.ShapeDtypeStruct((N, D_IN), jnp.float32),
    )(out_part, W_hg, b_hg.reshape(1, D_IN), W_lin, b_lin.reshape(1, D_IN))

    return y
